# baseline (speedup 1.0000x reference)
import functools

import jax
import jax.numpy as jnp
from jax import lax
from jax.experimental import pallas as pl
from jax.experimental.pallas import tpu as pltpu
from jax.experimental.pallas import tpu_sc as plsc

N_NODES = 10000
NP = 10240
C = 16
NSC = 2
NTILE = 16
NW = NSC * NTILE
CHUNK = 128
BURST = 16
ROWS_PER_TILE = NP // NTILE


def _sc_mesh():
    return plsc.VectorSubcoreMesh(core_axis_name="c", subcore_axis_name="s",
                                  num_cores=NSC)


def _fisr(x):
    i = lax.bitcast_convert_type(x, jnp.int32)
    i = jnp.int32(0x5F3759DF) - lax.shift_right_logical(i, 1)
    y = lax.bitcast_convert_type(i, jnp.float32)
    for _ in range(3):
        y = y * (1.5 - 0.5 * x * y * y)
    return y


def _zero_fill(buf, n):
    def fill(i, carry):
        buf[i, :] = jnp.zeros((C,), jnp.float32)
        return carry
    lax.fori_loop(0, n, fill, 0)


def _init_acc_zero(acc, zbuf, row0):
    _zero_fill(zbuf, CHUNK)
    for r in range(ROWS_PER_TILE // CHUNK):
        pltpu.sync_copy(zbuf, acc.at[pl.ds(row0 + r * CHUNK, CHUNK)])


def _edge_stream(nblocks, sidx_v, didx_v, rows_v, g_sh, acc, sems):
    def gather(blk, u):
        return pltpu.async_copy(g_sh.at[sidx_v.at[blk]], rows_v[u], sems[u])

    def scatter(blk, u):
        return pltpu.async_copy(rows_v[u], acc.at[didx_v.at[blk]],
                                sems[2 + u], add=True)

    g_descs = {0: gather(0, 0)}
    s_descs = {}
    for blk in range(nblocks):
        u = blk % 2
        g_descs[blk].wait()
        s_descs[blk] = scatter(blk, u)
        if blk + 1 < nblocks:
            if blk >= 1:
                s_descs[blk - 1].wait()
            g_descs[blk + 1] = gather(blk + 1, 1 - u)
    if nblocks >= 2:
        s_descs[nblocks - 2].wait()
    s_descs[nblocks - 1].wait()


def _sc_degree(nblocks):
    @functools.partial(
        pl.kernel,
        out_type=jax.ShapeDtypeStruct((NSC, NP), jnp.float32),
        mesh=_sc_mesh(),
        compiler_params=pltpu.CompilerParams(use_tc_tiling_on_sc=False),
        scratch_types=[
            pltpu.VMEM((nblocks, BURST * CHUNK), jnp.int32),
            pltpu.VMEM((BURST * CHUNK,), jnp.float32),
            pltpu.VMEM((ROWS_PER_TILE,), jnp.float32),
            pltpu.VMEM_SHARED((NP,), jnp.float32),
            pltpu.SemaphoreType.DMA,
        ],
    )
    def deg_kernel(dst_hbm, ones_hbm, out_hbm, didx_v, ones_v, zbuf, acc,
                   sem):
        cid = lax.axis_index("c")
        sid = lax.axis_index("s")
        wid = cid * NTILE + sid
        row0 = sid * ROWS_PER_TILE

        def zfill(i, carry):
            zbuf[pl.ds(i * 16, 16)] = jnp.zeros((16,), jnp.float32)
            return carry
        stage = [
            pltpu.async_copy(dst_hbm.at[wid], didx_v, sem),
            pltpu.async_copy(ones_hbm, ones_v, sem),
        ]
        lax.fori_loop(0, ROWS_PER_TILE // 16, zfill, 0)
        pltpu.sync_copy(zbuf, acc.at[pl.ds(row0, ROWS_PER_TILE)])
        for d in stage:
            d.wait()
        plsc.subcore_barrier()

        descs = [
            pltpu.async_copy(ones_v, acc.at[didx_v.at[blk]], sem, add=True)
            for blk in range(nblocks)
        ]
        for d in descs:
            d.wait()
        plsc.subcore_barrier()
        pltpu.sync_copy(acc.at[pl.ds(row0, ROWS_PER_TILE)],
                        out_hbm.at[cid, pl.ds(row0, ROWS_PER_TILE)])

    return deg_kernel


def _sc_round1(nblocks):
    @functools.partial(
        pl.kernel,
        out_type=(jax.ShapeDtypeStruct((NSC, NP, C), jnp.float32),
                  jax.ShapeDtypeStruct((NP, C), jnp.float32)),
        mesh=_sc_mesh(),
        compiler_params=pltpu.CompilerParams(use_tc_tiling_on_sc=False),
        scratch_types=[
            pltpu.VMEM((nblocks, BURST * CHUNK), jnp.int32),
            pltpu.VMEM((nblocks, BURST * CHUNK), jnp.int32),
            [pltpu.VMEM((BURST * CHUNK, C), jnp.float32) for _ in range(2)],
            pltpu.VMEM((2 * ROWS_PER_TILE,), jnp.float32),
            pltpu.VMEM_SHARED((NP, C), jnp.float32),
            pltpu.VMEM_SHARED((NP, C), jnp.float32),
            [pltpu.SemaphoreType.DMA for _ in range(4)],
        ],
    )
    def r1_kernel(y_hbm, dp_hbm, src_hbm, dst_hbm, q_hbm, g_hbm,
                  sidx_v, didx_v, rows_v, dp_v, acc, g_sh, sems):
        cid = lax.axis_index("c")
        sid = lax.axis_index("s")
        wid = cid * NTILE + sid
        row0 = sid * ROWS_PER_TILE
        half = ROWS_PER_TILE
        stage = [
            pltpu.async_copy(src_hbm.at[wid], sidx_v, sems[0]),
            pltpu.async_copy(dst_hbm.at[wid], didx_v, sems[0]),
            pltpu.async_copy(y_hbm.at[pl.ds(row0, half)],
                             rows_v[0].at[pl.ds(0, half)], sems[0]),
            pltpu.async_copy(dp_hbm.at[0, pl.ds(row0, half)],
                             dp_v.at[pl.ds(0, half)], sems[0]),
            pltpu.async_copy(dp_hbm.at[1, pl.ds(row0, half)],
                             dp_v.at[pl.ds(half, half)], sems[0]),
        ]
        _init_acc_zero(acc, rows_v[1].at[pl.ds(0, CHUNK)], row0)
        for d in stage:
            d.wait()

        def scale16(i16, carry):
            base = i16 * 16
            degv = (dp_v[pl.ds(base, 16)] + dp_v[pl.ds(half + base, 16)]
                    + 1.0)
            disv = _fisr(degv)
            for j in range(16):
                r = base + j
                rows_v[0][half + r, :] = rows_v[0][r, :] * disv[j]
            return carry
        lax.fori_loop(0, half // 16, scale16, 0)
        pltpu.sync_copy(rows_v[0].at[pl.ds(half, half)],
                        g_sh.at[pl.ds(row0, half)])
        pltpu.sync_copy(rows_v[0].at[pl.ds(half, half)],
                        g_hbm.at[pl.ds(row0, half)])
        plsc.subcore_barrier()

        _edge_stream(nblocks, sidx_v, didx_v, rows_v, g_sh, acc, sems)
        plsc.subcore_barrier()
        pltpu.sync_copy(acc.at[pl.ds(row0, ROWS_PER_TILE)],
                        q_hbm.at[cid, pl.ds(row0, ROWS_PER_TILE)])

    return r1_kernel


def _sc_round2(nblocks):
    @functools.partial(
        pl.kernel,
        out_type=(jax.ShapeDtypeStruct((NSC, NP, C), jnp.float32),
                  jax.ShapeDtypeStruct((NP, C), jnp.float32)),
        mesh=_sc_mesh(),
        compiler_params=pltpu.CompilerParams(use_tc_tiling_on_sc=False),
        scratch_types=[
            pltpu.VMEM((nblocks, BURST * CHUNK), jnp.int32),
            pltpu.VMEM((nblocks, BURST * CHUNK), jnp.int32),
            [pltpu.VMEM((BURST * CHUNK, C), jnp.float32) for _ in range(2)],
            pltpu.VMEM((ROWS_PER_TILE, C), jnp.float32),
            pltpu.VMEM((2 * ROWS_PER_TILE,), jnp.float32),
            pltpu.VMEM_SHARED((NP, C), jnp.float32),
            pltpu.VMEM_SHARED((NP, C), jnp.float32),
            [pltpu.SemaphoreType.DMA for _ in range(4)],
        ],
    )
    def r2_kernel(q_hbm, g_hbm, dp_hbm, src_hbm, dst_hbm, r_out_hbm,
                  t_out_hbm, sidx_v, didx_v, rows_v, aux_v, dp_v, acc, g_sh,
                  sems):
        cid = lax.axis_index("c")
        sid = lax.axis_index("s")
        wid = cid * NTILE + sid
        row0 = sid * ROWS_PER_TILE
        half = ROWS_PER_TILE
        stage = [
            pltpu.async_copy(src_hbm.at[wid], sidx_v, sems[0]),
            pltpu.async_copy(dst_hbm.at[wid], didx_v, sems[0]),
            pltpu.async_copy(q_hbm.at[0, pl.ds(row0, half)],
                             rows_v[0].at[pl.ds(0, half)], sems[0]),
            pltpu.async_copy(q_hbm.at[1, pl.ds(row0, half)],
                             rows_v[0].at[pl.ds(half, half)], sems[0]),
            pltpu.async_copy(g_hbm.at[pl.ds(row0, half)],
                             rows_v[1].at[pl.ds(0, half)], sems[0]),
            pltpu.async_copy(dp_hbm.at[0, pl.ds(row0, half)],
                             dp_v.at[pl.ds(0, half)], sems[0]),
            pltpu.async_copy(dp_hbm.at[1, pl.ds(row0, half)],
                             dp_v.at[pl.ds(half, half)], sems[0]),
        ]
        _init_acc_zero(acc, aux_v.at[pl.ds(0, CHUNK)], row0)
        for d in stage:
            d.wait()

        def comb16(i16, carry):
            base = i16 * 16
            degv = (dp_v[pl.ds(base, 16)] + dp_v[pl.ds(half + base, 16)]
                    + 1.0)
            rv = 1.0 / degv
            for j in range(16):
                r = base + j
                s = (rows_v[0][r, :] + rows_v[0][half + r, :]
                     + rows_v[1][r, :])
                aux_v[r, :] = s * rv[j]
            return carry
        lax.fori_loop(0, half // 16, comb16, 0)
        pltpu.sync_copy(aux_v, g_sh.at[pl.ds(row0, half)])
        pltpu.sync_copy(aux_v, t_out_hbm.at[pl.ds(row0, half)])
        plsc.subcore_barrier()

        _edge_stream(nblocks, sidx_v, didx_v, rows_v, g_sh, acc, sems)
        plsc.subcore_barrier()
        pltpu.sync_copy(acc.at[pl.ds(row0, ROWS_PER_TILE)],
                        r_out_hbm.at[cid, pl.ds(row0, ROWS_PER_TILE)])

    return r2_kernel


_BR = 1024


def _tc_matmul(x_pad, w):
    def body(x_ref, w_ref, y_ref):
        y_ref[...] = lax.dot_general(
            x_ref[...], w_ref[...], (((1,), (1,)), ((), ())),
            preferred_element_type=jnp.float32)

    return pl.pallas_call(
        body,
        grid=(NP // _BR,),
        in_specs=[
            pl.BlockSpec((_BR, 128), lambda i: (i, 0)),
            pl.BlockSpec((C, 128), lambda i: (0, 0)),
        ],
        out_specs=pl.BlockSpec((_BR, C), lambda i: (i, 0)),
        out_shape=jax.ShapeDtypeStruct((NP, C), jnp.float32),
    )(x_pad, w)


def _tc_final(rp, t, degp, b2):
    def body(r_ref, t_ref, dp_ref, b_ref, o_ref):
        deg = dp_ref[0]
        for i in range(1, NSC):
            deg = deg + dp_ref[i]
        deg = deg + 1.0
        s = r_ref[0]
        for i in range(1, NSC):
            s = s + r_ref[i]
        s = s + t_ref[...]
        o_ref[...] = s * lax.rsqrt(deg) + b_ref[...]

    return pl.pallas_call(
        body,
        grid=(NP // _BR,),
        in_specs=[
            pl.BlockSpec((NSC, _BR, C), lambda i: (0, i, 0)),
            pl.BlockSpec((_BR, C), lambda i: (i, 0)),
            pl.BlockSpec((NSC, _BR, 1), lambda i: (0, i, 0)),
            pl.BlockSpec((1, C), lambda i: (0, 0)),
        ],
        out_specs=pl.BlockSpec((_BR, C), lambda i: (i, 0)),
        out_shape=jax.ShapeDtypeStruct((NP, C), jnp.float32),
    )(rp, t, degp, b2)


def kernel(x, edge_index, W, b):
    E = edge_index.shape[1]
    blk_edges = BURST * CHUNK
    nblocks = -(-E // (NW * blk_edges))
    per_w = nblocks * blk_edges
    e_pad = NW * per_w

    ei = edge_index.astype(jnp.int32)
    pad = jnp.full((2, e_pad - E), N_NODES, jnp.int32)
    ei = jnp.concatenate([ei, pad], axis=1)
    src = ei[0].reshape(NW, nblocks, blk_edges)
    dst = ei[1].reshape(NW, nblocks, blk_edges)
    x_pad = jnp.pad(x, ((0, NP - x.shape[0]), (0, 0)))
    ones = jnp.ones((blk_edges,), jnp.float32)

    y = _tc_matmul(x_pad, W)
    degp = _sc_degree(nblocks)(dst, ones)
    qp, g = _sc_round1(nblocks)(y, degp, src, dst)
    rp, t = _sc_round2(nblocks)(qp, g, degp, src, dst)
    out = _tc_final(rp, t, degp.reshape(NSC, NP, 1), b.reshape(1, C))
    return out[:N_NODES]


# speedup vs baseline: 1.0103x; 1.0103x over previous
import functools

import jax
import jax.numpy as jnp
from jax import lax
from jax.experimental import pallas as pl
from jax.experimental.pallas import tpu as pltpu
from jax.experimental.pallas import tpu_sc as plsc

N_NODES = 10000
NP = 10240
C = 16
NSC = 2
NTILE = 16
NW = NSC * NTILE
CHUNK = 128
BURST = 10
ROWS_PER_TILE = NP // NTILE


def _sc_mesh():
    return plsc.VectorSubcoreMesh(core_axis_name="c", subcore_axis_name="s",
                                  num_cores=NSC)


def _fisr(x):
    i = lax.bitcast_convert_type(x, jnp.int32)
    i = jnp.int32(0x5F3759DF) - lax.shift_right_logical(i, 1)
    y = lax.bitcast_convert_type(i, jnp.float32)
    for _ in range(3):
        y = y * (1.5 - 0.5 * x * y * y)
    return y


def _zero_fill(buf, n):
    def fill(i, carry):
        buf[i, :] = jnp.zeros((C,), jnp.float32)
        return carry
    lax.fori_loop(0, n, fill, 0)


def _init_acc_zero(acc, zbuf, row0):
    _zero_fill(zbuf, CHUNK)
    for r in range(ROWS_PER_TILE // CHUNK):
        pltpu.sync_copy(zbuf, acc.at[pl.ds(row0 + r * CHUNK, CHUNK)])


def _edge_stream(nblocks, sidx_v, didx_v, rows_v, g_sh, acc, sems):
    nbuf = len(rows_v)

    def gather(blk, u):
        return pltpu.async_copy(g_sh.at[sidx_v.at[blk]], rows_v[u], sems[u])

    def scatter(blk, u):
        return pltpu.async_copy(rows_v[u], acc.at[didx_v.at[blk]],
                                sems[nbuf + u], add=True)

    g_descs = {}
    s_descs = {}
    for blk in range(min(nbuf - 1, nblocks)):
        g_descs[blk] = gather(blk, blk % nbuf)
    for blk in range(nblocks):
        g_descs[blk].wait()
        s_descs[blk] = scatter(blk, blk % nbuf)
        nxt = blk + nbuf - 1
        if nxt < nblocks:
            if nxt - nbuf >= 0:
                s_descs[nxt - nbuf].wait()
            g_descs[nxt] = gather(nxt, nxt % nbuf)
    for j in range(max(0, nblocks - nbuf), nblocks):
        s_descs[j].wait()


def _sc_degree(nblocks):
    @functools.partial(
        pl.kernel,
        out_type=jax.ShapeDtypeStruct((NSC, NP), jnp.float32),
        mesh=_sc_mesh(),
        compiler_params=pltpu.CompilerParams(use_tc_tiling_on_sc=False),
        scratch_types=[
            pltpu.VMEM((nblocks, BURST * CHUNK), jnp.int32),
            pltpu.VMEM((BURST * CHUNK,), jnp.float32),
            pltpu.VMEM((ROWS_PER_TILE,), jnp.float32),
            pltpu.VMEM_SHARED((NP,), jnp.float32),
            pltpu.SemaphoreType.DMA,
        ],
    )
    def deg_kernel(dst_hbm, ones_hbm, out_hbm, didx_v, ones_v, zbuf, acc,
                   sem):
        cid = lax.axis_index("c")
        sid = lax.axis_index("s")
        wid = cid * NTILE + sid
        row0 = sid * ROWS_PER_TILE

        def zfill(i, carry):
            zbuf[pl.ds(i * 16, 16)] = jnp.zeros((16,), jnp.float32)
            return carry
        stage = [
            pltpu.async_copy(dst_hbm.at[wid], didx_v, sem),
            pltpu.async_copy(ones_hbm, ones_v, sem),
        ]
        lax.fori_loop(0, ROWS_PER_TILE // 16, zfill, 0)
        pltpu.sync_copy(zbuf, acc.at[pl.ds(row0, ROWS_PER_TILE)])
        for d in stage:
            d.wait()
        plsc.subcore_barrier()

        descs = [
            pltpu.async_copy(ones_v, acc.at[didx_v.at[blk]], sem, add=True)
            for blk in range(nblocks)
        ]
        for d in descs:
            d.wait()
        plsc.subcore_barrier()
        pltpu.sync_copy(acc.at[pl.ds(row0, ROWS_PER_TILE)],
                        out_hbm.at[cid, pl.ds(row0, ROWS_PER_TILE)])

    return deg_kernel


def _sc_round1(nblocks):
    @functools.partial(
        pl.kernel,
        out_type=(jax.ShapeDtypeStruct((NSC, NP, C), jnp.float32),
                  jax.ShapeDtypeStruct((NP, C), jnp.float32)),
        mesh=_sc_mesh(),
        compiler_params=pltpu.CompilerParams(use_tc_tiling_on_sc=False),
        scratch_types=[
            pltpu.VMEM((nblocks, BURST * CHUNK), jnp.int32),
            pltpu.VMEM((nblocks, BURST * CHUNK), jnp.int32),
            [pltpu.VMEM((BURST * CHUNK, C), jnp.float32) for _ in range(2)],
            pltpu.VMEM((2 * ROWS_PER_TILE,), jnp.float32),
            pltpu.VMEM_SHARED((NP, C), jnp.float32),
            pltpu.VMEM_SHARED((NP, C), jnp.float32),
            [pltpu.SemaphoreType.DMA for _ in range(4)],
        ],
    )
    def r1_kernel(y_hbm, dp_hbm, src_hbm, dst_hbm, q_hbm, g_hbm,
                  sidx_v, didx_v, rows_v, dp_v, acc, g_sh, sems):
        cid = lax.axis_index("c")
        sid = lax.axis_index("s")
        wid = cid * NTILE + sid
        row0 = sid * ROWS_PER_TILE
        half = ROWS_PER_TILE
        stage = [
            pltpu.async_copy(src_hbm.at[wid], sidx_v, sems[0]),
            pltpu.async_copy(dst_hbm.at[wid], didx_v, sems[0]),
            pltpu.async_copy(y_hbm.at[pl.ds(row0, half)],
                             rows_v[0].at[pl.ds(0, half)], sems[0]),
            pltpu.async_copy(dp_hbm.at[0, pl.ds(row0, half)],
                             dp_v.at[pl.ds(0, half)], sems[0]),
            pltpu.async_copy(dp_hbm.at[1, pl.ds(row0, half)],
                             dp_v.at[pl.ds(half, half)], sems[0]),
        ]
        _init_acc_zero(acc, rows_v[1].at[pl.ds(0, CHUNK)], row0)
        for d in stage:
            d.wait()

        def scale16(i16, carry):
            base = i16 * 16
            degv = (dp_v[pl.ds(base, 16)] + dp_v[pl.ds(half + base, 16)]
                    + 1.0)
            disv = _fisr(degv)
            for j in range(16):
                r = base + j
                rows_v[0][half + r, :] = rows_v[0][r, :] * disv[j]
            return carry
        lax.fori_loop(0, half // 16, scale16, 0)
        pltpu.sync_copy(rows_v[0].at[pl.ds(half, half)],
                        g_sh.at[pl.ds(row0, half)])
        pltpu.sync_copy(rows_v[0].at[pl.ds(half, half)],
                        g_hbm.at[pl.ds(row0, half)])
        plsc.subcore_barrier()

        _edge_stream(nblocks, sidx_v, didx_v, rows_v, g_sh, acc, sems)
        plsc.subcore_barrier()
        pltpu.sync_copy(acc.at[pl.ds(row0, ROWS_PER_TILE)],
                        q_hbm.at[cid, pl.ds(row0, ROWS_PER_TILE)])

    return r1_kernel


def _sc_round2(nblocks):
    @functools.partial(
        pl.kernel,
        out_type=(jax.ShapeDtypeStruct((NSC, NP, C), jnp.float32),
                  jax.ShapeDtypeStruct((NP, C), jnp.float32)),
        mesh=_sc_mesh(),
        compiler_params=pltpu.CompilerParams(use_tc_tiling_on_sc=False),
        scratch_types=[
            pltpu.VMEM((nblocks, BURST * CHUNK), jnp.int32),
            pltpu.VMEM((nblocks, BURST * CHUNK), jnp.int32),
            [pltpu.VMEM((BURST * CHUNK, C), jnp.float32) for _ in range(3)],
            pltpu.VMEM((ROWS_PER_TILE, C), jnp.float32),
            pltpu.VMEM((2 * ROWS_PER_TILE,), jnp.float32),
            pltpu.VMEM_SHARED((NP, C), jnp.float32),
            pltpu.VMEM_SHARED((NP, C), jnp.float32),
            [pltpu.SemaphoreType.DMA for _ in range(6)],
        ],
    )
    def r2_kernel(q_hbm, g_hbm, dp_hbm, src_hbm, dst_hbm, r_out_hbm,
                  t_out_hbm, sidx_v, didx_v, rows_v, aux_v, dp_v, acc, g_sh,
                  sems):
        cid = lax.axis_index("c")
        sid = lax.axis_index("s")
        wid = cid * NTILE + sid
        row0 = sid * ROWS_PER_TILE
        half = ROWS_PER_TILE
        stage = [
            pltpu.async_copy(src_hbm.at[wid], sidx_v, sems[0]),
            pltpu.async_copy(dst_hbm.at[wid], didx_v, sems[0]),
            pltpu.async_copy(q_hbm.at[0, pl.ds(row0, half)],
                             rows_v[0].at[pl.ds(0, half)], sems[0]),
            pltpu.async_copy(q_hbm.at[1, pl.ds(row0, half)],
                             rows_v[0].at[pl.ds(half, half)], sems[0]),
            pltpu.async_copy(g_hbm.at[pl.ds(row0, half)],
                             rows_v[1].at[pl.ds(0, half)], sems[0]),
            pltpu.async_copy(dp_hbm.at[0, pl.ds(row0, half)],
                             dp_v.at[pl.ds(0, half)], sems[0]),
            pltpu.async_copy(dp_hbm.at[1, pl.ds(row0, half)],
                             dp_v.at[pl.ds(half, half)], sems[0]),
        ]
        _init_acc_zero(acc, aux_v.at[pl.ds(0, CHUNK)], row0)
        for d in stage:
            d.wait()

        def comb16(i16, carry):
            base = i16 * 16
            degv = (dp_v[pl.ds(base, 16)] + dp_v[pl.ds(half + base, 16)]
                    + 1.0)
            rv = 1.0 / degv
            for j in range(16):
                r = base + j
                s = (rows_v[0][r, :] + rows_v[0][half + r, :]
                     + rows_v[1][r, :])
                aux_v[r, :] = s * rv[j]
            return carry
        lax.fori_loop(0, half // 16, comb16, 0)
        pltpu.sync_copy(aux_v, g_sh.at[pl.ds(row0, half)])
        pltpu.sync_copy(aux_v, t_out_hbm.at[pl.ds(row0, half)])
        plsc.subcore_barrier()

        _edge_stream(nblocks, sidx_v, didx_v, rows_v, g_sh, acc, sems)
        plsc.subcore_barrier()
        pltpu.sync_copy(acc.at[pl.ds(row0, ROWS_PER_TILE)],
                        r_out_hbm.at[cid, pl.ds(row0, ROWS_PER_TILE)])

    return r2_kernel


_BR = 1024


def _tc_matmul(x_pad, w):
    def body(x_ref, w_ref, y_ref):
        y_ref[...] = lax.dot_general(
            x_ref[...], w_ref[...], (((1,), (1,)), ((), ())),
            preferred_element_type=jnp.float32)

    return pl.pallas_call(
        body,
        grid=(NP // _BR,),
        in_specs=[
            pl.BlockSpec((_BR, 128), lambda i: (i, 0)),
            pl.BlockSpec((C, 128), lambda i: (0, 0)),
        ],
        out_specs=pl.BlockSpec((_BR, C), lambda i: (i, 0)),
        out_shape=jax.ShapeDtypeStruct((NP, C), jnp.float32),
    )(x_pad, w)


def _tc_final(rp, t, degp, b2):
    def body(r_ref, t_ref, dp_ref, b_ref, o_ref):
        deg = dp_ref[0]
        for i in range(1, NSC):
            deg = deg + dp_ref[i]
        deg = deg + 1.0
        s = r_ref[0]
        for i in range(1, NSC):
            s = s + r_ref[i]
        s = s + t_ref[...]
        o_ref[...] = s * lax.rsqrt(deg) + b_ref[...]

    return pl.pallas_call(
        body,
        grid=(NP // _BR,),
        in_specs=[
            pl.BlockSpec((NSC, _BR, C), lambda i: (0, i, 0)),
            pl.BlockSpec((_BR, C), lambda i: (i, 0)),
            pl.BlockSpec((NSC, _BR, 1), lambda i: (0, i, 0)),
            pl.BlockSpec((1, C), lambda i: (0, 0)),
        ],
        out_specs=pl.BlockSpec((_BR, C), lambda i: (i, 0)),
        out_shape=jax.ShapeDtypeStruct((NP, C), jnp.float32),
    )(rp, t, degp, b2)


def kernel(x, edge_index, W, b):
    E = edge_index.shape[1]
    blk_edges = BURST * CHUNK
    nblocks = -(-E // (NW * blk_edges))
    per_w = nblocks * blk_edges
    e_pad = NW * per_w

    ei = edge_index.astype(jnp.int32)
    pad = jnp.full((2, e_pad - E), N_NODES, jnp.int32)
    ei = jnp.concatenate([ei, pad], axis=1)
    src = ei[0].reshape(NW, nblocks, blk_edges)
    dst = ei[1].reshape(NW, nblocks, blk_edges)
    x_pad = jnp.pad(x, ((0, NP - x.shape[0]), (0, 0)))
    ones = jnp.ones((blk_edges,), jnp.float32)

    y = _tc_matmul(x_pad, W)
    degp = _sc_degree(nblocks)(dst, ones)
    qp, g = _sc_round1(nblocks)(y, degp, src, dst)
    rp, t = _sc_round2(nblocks)(qp, g, degp, src, dst)
    out = _tc_final(rp, t, degp.reshape(NSC, NP, 1), b.reshape(1, C))
    return out[:N_NODES]
